# baseline (speedup 1.0000x reference)
import functools

import jax
import jax.numpy as jnp
from jax import lax
from jax.experimental import pallas as pl
from jax.experimental.pallas import tpu as pltpu
from jax.experimental.pallas import tpu_sc as plsc

STATE_LEN = 1024
SEQ_LEN = 2048
NPOI = 4096

NUM_CORES = 2
NUM_SUBCORES = 16
LANES = 16
NW = NUM_CORES * NUM_SUBCORES
ROWS_PER_W = STATE_LEN // NW
CHUNK = 8
NCHUNK = ROWS_PER_W // CHUNK
JSTEPS = SEQ_LEN // LANES


def _body(his_hbm, cur_hbm, d_hbm, out_hbm, his_v, cur_v, rows_v, en_v, sem):
    wid = lax.axis_index("s") * NUM_CORES + lax.axis_index("c")
    base = wid * ROWS_PER_W

    pltpu.sync_copy(his_hbm, his_v)
    pltpu.sync_copy(cur_hbm.at[pl.ds(base, ROWS_PER_W)], cur_v)

    for c in range(NCHUNK):
        pltpu.async_copy(
            d_hbm.at[cur_v.at[pl.ds(c * CHUNK, CHUNK)]], rows_v, sem
        ).wait()

        for r in range(CHUNK):
            row_ids = jnp.full((LANES,), r, jnp.int32)

            def pass1(j, m, r=r, row_ids=row_ids):
                idx = his_v[pl.ds(j * LANES, LANES)]
                v = plsc.load_gather(rows_v, [row_ids, idx])
                nz = v != 0.0
                e = jnp.where(nz, 1.0 / jnp.where(nz, v, 1.0), 1e-6)
                en_v[r, pl.ds(j * LANES, LANES)] = e
                return jnp.maximum(m, e)

            m16 = lax.fori_loop(
                0, JSTEPS, pass1, jnp.full((LANES,), -jnp.inf, jnp.float32)
            )
            row_max = jnp.max(m16)

            def pass2(j, s, r=r):
                e = en_v[r, pl.ds(j * LANES, LANES)]
                p = jnp.exp(e - row_max)
                en_v[r, pl.ds(j * LANES, LANES)] = p
                return s + p

            s16 = lax.fori_loop(0, JSTEPS, pass2, jnp.zeros((LANES,), jnp.float32))
            inv16 = jnp.full((LANES,), 1.0, jnp.float32) / jnp.broadcast_to(
                jnp.sum(s16), (LANES,)
            )

            def pass3(j, carry, r=r):
                en_v[r, pl.ds(j * LANES, LANES)] = (
                    en_v[r, pl.ds(j * LANES, LANES)] * inv16
                )
                return carry

            lax.fori_loop(0, JSTEPS, pass3, 0)

        pltpu.sync_copy(en_v, out_hbm.at[pl.ds(base + c * CHUNK, CHUNK)])


@jax.jit
def kernel(his, cur, poi_distance_mat):
    run = pl.kernel(
        _body,
        out_type=jax.ShapeDtypeStruct((STATE_LEN, SEQ_LEN), jnp.float32),
        mesh=plsc.VectorSubcoreMesh(core_axis_name="c", subcore_axis_name="s"),
        scratch_types=[
            pltpu.VMEM((SEQ_LEN,), jnp.int32),
            pltpu.VMEM((ROWS_PER_W,), jnp.int32),
            pltpu.VMEM((CHUNK, NPOI), jnp.float32),
            pltpu.VMEM((CHUNK, SEQ_LEN), jnp.float32),
            pltpu.SemaphoreType.DMA,
        ],
        compiler_params=pltpu.CompilerParams(
            use_tc_tiling_on_sc=False, needs_layout_passes=False
        ),
    )
    return run(his.astype(jnp.int32), cur.astype(jnp.int32), poi_distance_mat)


# speedup vs baseline: 1.3723x; 1.3723x over previous
import jax
import jax.numpy as jnp
from jax import lax
from jax.experimental import pallas as pl
from jax.experimental.pallas import tpu as pltpu
from jax.experimental.pallas import tpu_sc as plsc

STATE_LEN = 1024
SEQ_LEN = 2048
NPOI = 4096

NUM_CORES = 2
NUM_SUBCORES = 16
LANES = 16
NW = NUM_CORES * NUM_SUBCORES
ROWS_PER_W = STATE_LEN // NW
CHUNK = 8
NCHUNK = ROWS_PER_W // CHUNK
UNROLL = 4
JSTEPS = SEQ_LEN // (LANES * UNROLL)


def _body(his_hbm, cur_hbm, d_hbm, out_hbm,
          his_v, cur_v, rows_a, rows_b, en_v, sem_a, sem_b):
    wid = lax.axis_index("s") * NUM_CORES + lax.axis_index("c")
    base = wid * ROWS_PER_W

    pltpu.sync_copy(his_hbm, his_v)
    pltpu.sync_copy(cur_hbm.at[pl.ds(base, ROWS_PER_W)], cur_v)

    bufs = (rows_a, rows_b)
    sems = (sem_a, sem_b)

    def start_gather(c):
        return pltpu.async_copy(
            d_hbm.at[cur_v.at[pl.ds(c * CHUNK, CHUNK)]],
            bufs[c % 2], sems[c % 2],
        )

    pending = start_gather(0)
    for c in range(NCHUNK):
        pending.wait()
        if c + 1 < NCHUNK:
            pending = start_gather(c + 1)
        rows_v = bufs[c % 2]

        for r in range(CHUNK):
            row_ids = jnp.full((LANES,), r, jnp.int32)

            def pass1(j, m, r=r, row_ids=row_ids, rows_v=rows_v):
                for u in range(UNROLL):
                    off = (j * UNROLL + u) * LANES
                    idx = his_v[pl.ds(off, LANES)]
                    v = plsc.load_gather(rows_v, [row_ids, idx])
                    e = jnp.where(v != 0.0, 1.0 / v, 1e-6)
                    en_v[r, pl.ds(off, LANES)] = e
                    m = jnp.maximum(m, e)
                return m

            m16 = lax.fori_loop(
                0, JSTEPS, pass1, jnp.full((LANES,), -jnp.inf, jnp.float32)
            )
            row_max = jnp.max(m16)

            def pass2(j, s, r=r):
                for u in range(UNROLL):
                    off = (j * UNROLL + u) * LANES
                    p = jnp.exp(en_v[r, pl.ds(off, LANES)] - row_max)
                    en_v[r, pl.ds(off, LANES)] = p
                    s = s + p
                return s

            s16 = lax.fori_loop(0, JSTEPS, pass2, jnp.zeros((LANES,), jnp.float32))
            inv16 = jnp.full((LANES,), 1.0, jnp.float32) / jnp.broadcast_to(
                jnp.sum(s16), (LANES,)
            )

            def pass3(j, carry, r=r):
                for u in range(UNROLL):
                    off = (j * UNROLL + u) * LANES
                    en_v[r, pl.ds(off, LANES)] = (
                        en_v[r, pl.ds(off, LANES)] * inv16
                    )
                return carry

            lax.fori_loop(0, JSTEPS, pass3, 0)

        pltpu.sync_copy(en_v, out_hbm.at[pl.ds(base + c * CHUNK, CHUNK)])


@jax.jit
def kernel(his, cur, poi_distance_mat):
    run = pl.kernel(
        _body,
        out_type=jax.ShapeDtypeStruct((STATE_LEN, SEQ_LEN), jnp.float32),
        mesh=plsc.VectorSubcoreMesh(core_axis_name="c", subcore_axis_name="s"),
        scratch_types=[
            pltpu.VMEM((SEQ_LEN,), jnp.int32),
            pltpu.VMEM((ROWS_PER_W,), jnp.int32),
            pltpu.VMEM((CHUNK, NPOI), jnp.float32),
            pltpu.VMEM((CHUNK, NPOI), jnp.float32),
            pltpu.VMEM((CHUNK, SEQ_LEN), jnp.float32),
            pltpu.SemaphoreType.DMA,
            pltpu.SemaphoreType.DMA,
        ],
        compiler_params=pltpu.CompilerParams(
            use_tc_tiling_on_sc=False, needs_layout_passes=False
        ),
    )
    return run(his.astype(jnp.int32), cur.astype(jnp.int32), poi_distance_mat)


# speedup vs baseline: 1.4706x; 1.0716x over previous
import jax
import jax.numpy as jnp
from jax import lax
from jax.experimental import pallas as pl
from jax.experimental.pallas import tpu as pltpu
from jax.experimental.pallas import tpu_sc as plsc

STATE_LEN = 1024
SEQ_LEN = 2048
NPOI = 4096

NUM_CORES = 2
NUM_SUBCORES = 16
LANES = 16
NW = NUM_CORES * NUM_SUBCORES
ROWS_PER_W = STATE_LEN // NW
CHUNK = 8
NCHUNK = ROWS_PER_W // CHUNK
U1 = 2
J1 = SEQ_LEN // (LANES * U1)
U23 = 8
J23 = SEQ_LEN // (LANES * U23)


def _body(his_hbm, cur_hbm, d_hbm, out_hbm,
          his_v, cur_v, rows_a, rows_b, en_v, sem_a, sem_b):
    wid = lax.axis_index("s") * NUM_CORES + lax.axis_index("c")
    base = wid * ROWS_PER_W

    pltpu.sync_copy(his_hbm, his_v)
    pltpu.sync_copy(cur_hbm.at[pl.ds(base, ROWS_PER_W)], cur_v)

    bufs = (rows_a, rows_b)
    sems = (sem_a, sem_b)

    def start_gather(c):
        return pltpu.async_copy(
            d_hbm.at[cur_v.at[pl.ds(c * CHUNK, CHUNK)]],
            bufs[c % 2], sems[c % 2],
        )

    row_ids = [jnp.full((LANES,), r, jnp.int32) for r in range(CHUNK)]
    neg_inf = jnp.full((LANES,), -jnp.inf, jnp.float32)

    pending = start_gather(0)
    for c in range(NCHUNK):
        pending.wait()
        if c + 1 < NCHUNK:
            pending = start_gather(c + 1)
        rows_v = bufs[c % 2]

        def pass1(j, ms, rows_v=rows_v):
            ms = list(ms)
            for u in range(U1):
                off = (j * U1 + u) * LANES
                idx = his_v[pl.ds(off, LANES)]
                for r in range(CHUNK):
                    v = plsc.load_gather(rows_v, [row_ids[r], idx])
                    e = jnp.where(v != 0.0, 1.0 / v, 1e-6)
                    en_v[r, pl.ds(off, LANES)] = e
                    ms[r] = jnp.maximum(ms[r], e)
            return tuple(ms)

        ms = lax.fori_loop(0, J1, pass1, tuple([neg_inf] * CHUNK))

        for r in range(CHUNK):
            row_max = jnp.max(ms[r])

            def pass2(j, s, r=r, row_max=row_max):
                for u in range(U23):
                    off = (j * U23 + u) * LANES
                    p = jnp.exp(en_v[r, pl.ds(off, LANES)] - row_max)
                    en_v[r, pl.ds(off, LANES)] = p
                    s = s + p
                return s

            s16 = lax.fori_loop(0, J23, pass2,
                                jnp.zeros((LANES,), jnp.float32))
            inv16 = jnp.full((LANES,), 1.0, jnp.float32) / jnp.broadcast_to(
                jnp.sum(s16), (LANES,)
            )

            def pass3(j, carry2, r=r, inv16=inv16):
                for u in range(U23):
                    off = (j * U23 + u) * LANES
                    en_v[r, pl.ds(off, LANES)] = (
                        en_v[r, pl.ds(off, LANES)] * inv16
                    )
                return carry2

            lax.fori_loop(0, J23, pass3, 0)

        pltpu.sync_copy(en_v, out_hbm.at[pl.ds(base + c * CHUNK, CHUNK)])


@jax.jit
def kernel(his, cur, poi_distance_mat):
    run = pl.kernel(
        _body,
        out_type=jax.ShapeDtypeStruct((STATE_LEN, SEQ_LEN), jnp.float32),
        mesh=plsc.VectorSubcoreMesh(core_axis_name="c", subcore_axis_name="s"),
        scratch_types=[
            pltpu.VMEM((SEQ_LEN,), jnp.int32),
            pltpu.VMEM((ROWS_PER_W,), jnp.int32),
            pltpu.VMEM((CHUNK, NPOI), jnp.float32),
            pltpu.VMEM((CHUNK, NPOI), jnp.float32),
            pltpu.VMEM((CHUNK, SEQ_LEN), jnp.float32),
            pltpu.SemaphoreType.DMA,
            pltpu.SemaphoreType.DMA,
        ],
        compiler_params=pltpu.CompilerParams(
            use_tc_tiling_on_sc=False, needs_layout_passes=False
        ),
    )
    return run(his.astype(jnp.int32), cur.astype(jnp.int32), poi_distance_mat)


# speedup vs baseline: 1.9043x; 1.2949x over previous
import jax
import jax.numpy as jnp
from jax import lax
from jax.experimental import pallas as pl
from jax.experimental.pallas import tpu as pltpu
from jax.experimental.pallas import tpu_sc as plsc

STATE_LEN = 1024
SEQ_LEN = 2048
NPOI = 4096

NUM_CORES = 2
NUM_SUBCORES = 16
LANES = 16
NW = NUM_CORES * NUM_SUBCORES
ROWS_PER_W = STATE_LEN // NW
CHUNK = 8
NCHUNK = ROWS_PER_W // CHUNK
U1 = 4
J1 = SEQ_LEN // (LANES * U1)

TC_BLOCK_ROWS = 256


def _gather_body(his_hbm, cur_hbm, d_hbm, g_hbm,
                 his_v, cur_v, rows_a, rows_b, g_v, sem_a, sem_b):
    wid = lax.axis_index("s") * NUM_CORES + lax.axis_index("c")
    base = wid * ROWS_PER_W

    pltpu.sync_copy(his_hbm, his_v)
    pltpu.sync_copy(cur_hbm.at[pl.ds(base, ROWS_PER_W)], cur_v)

    bufs = (rows_a, rows_b)
    sems = (sem_a, sem_b)

    def start_gather(c):
        return pltpu.async_copy(
            d_hbm.at[cur_v.at[pl.ds(c * CHUNK, CHUNK)]],
            bufs[c % 2], sems[c % 2],
        )

    row_ids = [jnp.full((LANES,), r, jnp.int32) for r in range(CHUNK)]

    pending = start_gather(0)
    for c in range(NCHUNK):
        pending.wait()
        if c + 1 < NCHUNK:
            pending = start_gather(c + 1)
        rows_v = bufs[c % 2]

        def colgather(j, carry, rows_v=rows_v):
            for u in range(U1):
                off = (j * U1 + u) * LANES
                idx = his_v[pl.ds(off, LANES)]
                for r in range(CHUNK):
                    g_v[r, pl.ds(off, LANES)] = plsc.load_gather(
                        rows_v, [row_ids[r], idx]
                    )
            return carry

        lax.fori_loop(0, J1, colgather, 0)
        pltpu.sync_copy(g_v, g_hbm.at[pl.ds(base + c * CHUNK, CHUNK)])


def _softmax_body(g_ref, o_ref):
    d = g_ref[...]
    nz = d != 0.0
    e = jnp.where(nz, 1.0 / jnp.where(nz, d, 1.0), 1e-6)
    m = jnp.max(e, axis=-1, keepdims=True)
    p = jnp.exp(e - m)
    o_ref[...] = p / jnp.sum(p, axis=-1, keepdims=True)


@jax.jit
def kernel(his, cur, poi_distance_mat):
    gather = pl.kernel(
        _gather_body,
        out_type=jax.ShapeDtypeStruct((STATE_LEN, SEQ_LEN), jnp.float32),
        mesh=plsc.VectorSubcoreMesh(core_axis_name="c", subcore_axis_name="s"),
        scratch_types=[
            pltpu.VMEM((SEQ_LEN,), jnp.int32),
            pltpu.VMEM((ROWS_PER_W,), jnp.int32),
            pltpu.VMEM((CHUNK, NPOI), jnp.float32),
            pltpu.VMEM((CHUNK, NPOI), jnp.float32),
            pltpu.VMEM((CHUNK, SEQ_LEN), jnp.float32),
            pltpu.SemaphoreType.DMA,
            pltpu.SemaphoreType.DMA,
        ],
        compiler_params=pltpu.CompilerParams(
            use_tc_tiling_on_sc=False, needs_layout_passes=False
        ),
    )
    g = gather(his.astype(jnp.int32), cur.astype(jnp.int32), poi_distance_mat)

    softmax = pl.pallas_call(
        _softmax_body,
        grid=(STATE_LEN // TC_BLOCK_ROWS,),
        in_specs=[
            pl.BlockSpec((TC_BLOCK_ROWS, SEQ_LEN), lambda i: (i, 0)),
        ],
        out_specs=pl.BlockSpec((TC_BLOCK_ROWS, SEQ_LEN), lambda i: (i, 0)),
        out_shape=jax.ShapeDtypeStruct((STATE_LEN, SEQ_LEN), jnp.float32),
    )
    return softmax(g)
